# baseline (speedup 1.0000x reference)
import functools

import jax
import jax.numpy as jnp
from jax import lax
from jax.experimental import pallas as pl
from jax.experimental.pallas import tpu as pltpu
from jax.experimental.pallas import tpu_sc as plsc

_GATHER_DNUMS = lax.GatherDimensionNumbers(
    offset_dims=(), collapsed_slice_dims=(0,), start_index_map=(0,))


def _shuffle(t, idx):
    return lax.gather(t, idx[:, None], _GATHER_DNUMS, slice_sizes=(1,),
                      mode=lax.GatherScatterMode.PROMISE_IN_BOUNDS)


D = 128
L = 16
CHUNK = 80
NC, NS = 2, 16
NW = NC * NS


def _make_sc_call(E, N):
    n_chunks = E // CHUNK
    cpw = n_chunks // NW
    npairs = cpw // 2
    mesh = plsc.VectorSubcoreMesh(core_axis_name="c", subcore_axis_name="s")

    @functools.partial(
        pl.kernel,
        out_type=jax.ShapeDtypeStruct((NW, cpw, CHUNK), jnp.float32),
        mesh=mesh,
        scratch_types=[
            pltpu.VMEM((cpw, CHUNK), jnp.int32),
            pltpu.VMEM((cpw, CHUNK), jnp.int32),
            pltpu.VMEM((CHUNK, D), jnp.float32),
            pltpu.VMEM((CHUNK, D), jnp.float32),
            pltpu.VMEM((CHUNK, D), jnp.float32),
            pltpu.VMEM((CHUNK, D), jnp.float32),
            pltpu.VMEM((cpw, CHUNK), jnp.float32),
            pltpu.SemaphoreType.DMA,
            pltpu.SemaphoreType.DMA,
        ],
    )
    def sc_call(z_hbm, src_hbm, dst_hbm, out_hbm, idx_s, idx_d,
                rs_a, rd_a, rs_b, rd_b, out_v, sem_a, sem_b):
        wid = lax.axis_index("s") * NC + lax.axis_index("c")
        lane = lax.broadcasted_iota(jnp.int32, (L,), 0)

        pltpu.sync_copy(src_hbm.at[wid], idx_s)
        pltpu.sync_copy(dst_hbm.at[wid], idx_d)

        def fire(ci, rs, rd, sem):
            pltpu.async_copy(z_hbm.at[idx_s.at[ci]], rs, sem)
            pltpu.async_copy(z_hbm.at[idx_d.at[ci]], rd, sem)

        def drain(ci, rs, rd, sem):
            pltpu.make_async_copy(z_hbm.at[idx_s.at[ci]], rs, sem).wait()
            pltpu.make_async_copy(z_hbm.at[idx_d.at[ci]], rd, sem).wait()

        bits = [((lane >> s) & 1) == 1 for s in range(4)]
        shufs = [lane ^ (1 << s) for s in range(4)]

        def merge(a, b, s):
            u = jnp.where(bits[s], _shuffle(b, shufs[s]), a)
            v = jnp.where(bits[s], b, _shuffle(a, shufs[s]))
            return u + v

        def compute(ci, rs, rd):
            def edge_dot(e):
                t0 = rs[e, pl.ds(0, L)] * rd[e, pl.ds(0, L)]
                t1 = rs[e, pl.ds(L, L)] * rd[e, pl.ds(L, L)]
                for k in range(2, D // L, 2):
                    t0 = t0 + rs[e, pl.ds(k * L, L)] * rd[e, pl.ds(k * L, L)]
                    t1 = t1 + (rs[e, pl.ds((k + 1) * L, L)]
                               * rd[e, pl.ds((k + 1) * L, L)])
                return t0 + t1

            def quad(e):
                m0 = merge(edge_dot(e), edge_dot(e + 1), 0)
                m1 = merge(edge_dot(e + 2), edge_dot(e + 3), 0)
                return merge(m0, m1, 1)

            def group_body(g, _):
                e0 = g * L
                h0 = merge(quad(e0), quad(e0 + 4), 2)
                h1 = merge(quad(e0 + 8), quad(e0 + 12), 2)
                acc = merge(h0, h1, 3)
                out_v[ci, pl.ds(e0, L)] = 1.0 / (1.0 + jnp.exp(-acc))
                return 0

            lax.fori_loop(0, CHUNK // L, group_body, 0)

        fire(0, rs_a, rd_a, sem_a)

        def pair_body(pi, _):
            ca = 2 * pi
            cb = ca + 1
            fire(cb, rs_b, rd_b, sem_b)
            drain(ca, rs_a, rd_a, sem_a)
            compute(ca, rs_a, rd_a)
            fire(ca + 2, rs_a, rd_a, sem_a)
            drain(cb, rs_b, rd_b, sem_b)
            compute(cb, rs_b, rd_b)
            return 0

        lax.fori_loop(0, npairs, pair_body, 0)
        drain(cpw - 1, rs_a, rd_a, sem_a)
        compute(cpw - 1, rs_a, rd_a)

        pltpu.sync_copy(out_v, out_hbm.at[wid])

    return sc_call


def kernel(z, edge_index):
    E = edge_index.shape[1]
    ei = edge_index.astype(jnp.int32)
    cpw = E // CHUNK // NW
    src3d = ei[0].reshape(NW, cpw, CHUNK)
    dst3d = ei[1].reshape(NW, cpw, CHUNK)
    out3d = _make_sc_call(E, z.shape[0])(z, src3d, dst3d)
    return out3d.reshape(E)


# speedup vs baseline: 1.2006x; 1.2006x over previous
import functools

import jax
import jax.numpy as jnp
from jax import lax
from jax.experimental import pallas as pl
from jax.experimental.pallas import tpu as pltpu
from jax.experimental.pallas import tpu_sc as plsc

_GATHER_DNUMS = lax.GatherDimensionNumbers(
    offset_dims=(), collapsed_slice_dims=(0,), start_index_map=(0,))


def _shuffle(t, idx):
    return lax.gather(t, idx[:, None], _GATHER_DNUMS, slice_sizes=(1,),
                      mode=lax.GatherScatterMode.PROMISE_IN_BOUNDS)


D = 128
L = 16
CHUNK = 80
NC, NS = 2, 16
NW = NC * NS


def _make_sc_call(E, N):
    n_chunks = E // CHUNK
    cpw = n_chunks // NW
    npairs = cpw // 2
    mesh = plsc.VectorSubcoreMesh(core_axis_name="c", subcore_axis_name="s")

    @functools.partial(
        pl.kernel,
        out_type=jax.ShapeDtypeStruct((NW, cpw, CHUNK), jnp.float32),
        mesh=mesh,
        scratch_types=[
            pltpu.VMEM((cpw, CHUNK), jnp.int32),
            pltpu.VMEM((cpw, CHUNK), jnp.int32),
            pltpu.VMEM((CHUNK, D), jnp.float32),
            pltpu.VMEM((CHUNK, D), jnp.float32),
            pltpu.VMEM((CHUNK, D), jnp.float32),
            pltpu.VMEM((CHUNK, D), jnp.float32),
            pltpu.VMEM((cpw, CHUNK), jnp.float32),
            pltpu.SemaphoreType.DMA,
            pltpu.SemaphoreType.DMA,
        ],
    )
    def sc_call(z_hbm, src_hbm, dst_hbm, out_hbm, idx_s, idx_d,
                rs_a, rd_a, rs_b, rd_b, out_v, sem_a, sem_b):
        wid = lax.axis_index("s") * NC + lax.axis_index("c")
        lane = lax.broadcasted_iota(jnp.int32, (L,), 0)

        pltpu.sync_copy(src_hbm.at[wid], idx_s)
        pltpu.sync_copy(dst_hbm.at[wid], idx_d)

        def fire(ci, rs, rd, sem):
            pltpu.async_copy(z_hbm.at[idx_s.at[ci]], rs, sem)
            pltpu.async_copy(z_hbm.at[idx_d.at[ci]], rd, sem)

        def drain(ci, rs, rd, sem):
            pltpu.make_async_copy(z_hbm.at[idx_s.at[ci]], rs, sem).wait()
            pltpu.make_async_copy(z_hbm.at[idx_d.at[ci]], rd, sem).wait()

        bits = [((lane >> s) & 1) == 1 for s in range(4)]
        shufs = [lane ^ (1 << s) for s in range(4)]

        def merge(a, b, s):
            u = jnp.where(bits[s], _shuffle(b, shufs[s]), a)
            v = jnp.where(bits[s], b, _shuffle(a, shufs[s]))
            return u + v

        def compute(ci, rs, rd):
            def edge_dot(e):
                t = jnp.zeros((L,), jnp.float32)
                for k in range(D // L):
                    t = t + rs[e, pl.ds(k * L, L)] * rd[e, pl.ds(k * L, L)]
                return t

            def quad(e):
                m0 = merge(edge_dot(e), edge_dot(e + 1), 0)
                m1 = merge(edge_dot(e + 2), edge_dot(e + 3), 0)
                return merge(m0, m1, 1)

            def group_body(g, _):
                e0 = g * L
                h0 = merge(quad(e0), quad(e0 + 4), 2)
                h1 = merge(quad(e0 + 8), quad(e0 + 12), 2)
                acc = merge(h0, h1, 3)
                out_v[ci, pl.ds(e0, L)] = 1.0 / (1.0 + jnp.exp(-acc))
                return 0

            lax.fori_loop(0, CHUNK // L, group_body, 0)

        fire(0, rs_a, rd_a, sem_a)

        def pair_body(pi, _):
            ca = 2 * pi
            cb = ca + 1
            fire(cb, rs_b, rd_b, sem_b)
            drain(ca, rs_a, rd_a, sem_a)
            compute(ca, rs_a, rd_a)
            fire(ca + 2, rs_a, rd_a, sem_a)
            drain(cb, rs_b, rd_b, sem_b)
            compute(cb, rs_b, rd_b)
            return 0

        lax.fori_loop(0, npairs, pair_body, 0)
        drain(cpw - 1, rs_a, rd_a, sem_a)
        compute(cpw - 1, rs_a, rd_a)

        pltpu.sync_copy(out_v, out_hbm.at[wid])

    return sc_call


def kernel(z, edge_index):
    E = edge_index.shape[1]
    ei = edge_index.astype(jnp.int32)
    cpw = E // CHUNK // NW
    src3d = ei[0].reshape(NW, cpw, CHUNK)
    dst3d = ei[1].reshape(NW, cpw, CHUNK)
    out3d = _make_sc_call(E, z.shape[0])(z, src3d, dst3d)
    return out3d.reshape(E)
